# Initial kernel scaffold; baseline (speedup 1.0000x reference)
import jax
import jax.numpy as jnp
from jax.experimental import pallas as pl


def kernel(nodes_embed, adj, Win_w, Win_b, Wout_w, Wout_b, Wing_w, Wing_b, Woutg_w, Woutg_b, Rin_w, Rin_b, Rout_w, Rout_b, Ring_w, Ring_b, Routg_w, Routg_b):
    raise NotImplementedError("write your pallas kernel here")


# speedup vs baseline: 2.1959x; 2.1959x over previous
import jax
import jax.numpy as jnp
from jax.experimental import pallas as pl
from jax.experimental.pallas import tpu as pltpu

B, L, D = 4, 512, 140
U, T, NB = 4, 10, 2


def _gcnn_kernel(nodes_ref, adj_ref, wio_ref, bio_ref, wg_ref, bg_ref, out_ref):
    h = nodes_ref[0]
    adjb = adj_ref[0]

    adj_rare = adjb[U]
    for t in range(U + 1, T):
        adj_rare = adj_rare + adjb[t]

    for l in range(NB):
        p = jnp.dot(h, wio_ref[l], preferred_element_type=jnp.float32) + bio_ref[l]
        g = jax.nn.sigmoid(
            jnp.dot(h, wg_ref[l], preferred_element_type=jnp.float32) + bg_ref[l]
        )

        rin = p[:, 2 * U * D:2 * U * D + D] * g[:, 2 * U:2 * U + 1]
        rout = p[:, 2 * U * D + D:2 * U * D + 2 * D] * g[:, 2 * U + 1:2 * U + 2]

        acc_in = jnp.dot(adj_rare, rin, preferred_element_type=jnp.float32)
        acc_out = jax.lax.dot_general(
            adj_rare, rout, (((0,), (0,)), ((), ())),
            preferred_element_type=jnp.float32)

        for t in range(U):
            hin_t = p[:, t * D:(t + 1) * D] * g[:, t:t + 1]
            hout_t = p[:, (U + t) * D:(U + t + 1) * D] * g[:, U + t:U + t + 1]
            acc_in = acc_in + jnp.dot(adjb[t], hin_t,
                                      preferred_element_type=jnp.float32)
            acc_out = acc_out + jax.lax.dot_general(
                adjb[t], hout_t, (((0,), (0,)), ((), ())),
                preferred_element_type=jnp.float32)

        h = jnp.maximum(acc_in + acc_out, 0.0) + h

    out_ref[0] = h


def kernel(nodes_embed, adj, Win_w, Win_b, Wout_w, Wout_b, Wing_w, Wing_b,
           Woutg_w, Woutg_b, Rin_w, Rin_b, Rout_w, Rout_b, Ring_w, Ring_b,
           Routg_w, Routg_b):
    wio = jnp.concatenate([Win_w, Wout_w, Rin_w, Rout_w], axis=2)
    bio = jnp.concatenate([Win_b, Wout_b, Rin_b, Rout_b], axis=1)
    wg = jnp.concatenate([Wing_w, Woutg_w, Ring_w, Routg_w], axis=2)
    bg = jnp.concatenate([Wing_b, Woutg_b, Ring_b, Routg_b], axis=1)

    return pl.pallas_call(
        _gcnn_kernel,
        grid=(B,),
        in_specs=[
            pl.BlockSpec((1, L, D), lambda b: (b, 0, 0)),
            pl.BlockSpec((1, T, L, L), lambda b: (b, 0, 0, 0)),
            pl.BlockSpec((NB, D, 2 * U * D + 2 * D), lambda b: (0, 0, 0)),
            pl.BlockSpec((NB, 2 * U * D + 2 * D), lambda b: (0, 0)),
            pl.BlockSpec((NB, D, 2 * U + 2), lambda b: (0, 0, 0)),
            pl.BlockSpec((NB, 2 * U + 2), lambda b: (0, 0)),
        ],
        out_specs=pl.BlockSpec((1, L, D), lambda b: (b, 0, 0)),
        out_shape=jax.ShapeDtypeStruct((B, L, D), jnp.float32),
        compiler_params=pltpu.CompilerParams(
            dimension_semantics=("arbitrary",),
            vmem_limit_bytes=100 * 1024 * 1024,
        ),
    )(nodes_embed, adj, wio, bio, wg, bg)


# speedup vs baseline: 2.2432x; 1.0216x over previous
import jax
import jax.numpy as jnp
from jax.experimental import pallas as pl
from jax.experimental.pallas import tpu as pltpu

B, L, D = 4, 512, 140
U, T, NB = 4, 10, 2


def _gcnn_kernel(nodes_ref, adj_ref, wio_ref, bio_ref, wg_ref, bg_ref, out_ref):
    h = nodes_ref[0]
    adjb = adj_ref[0]

    adj_rare = adjb[U]
    for t in range(U + 1, T):
        adj_rare = adj_rare + adjb[t]

    for l in range(NB):
        p = jnp.dot(h, wio_ref[l], preferred_element_type=jnp.float32) + bio_ref[l]
        g = jax.nn.sigmoid(
            jnp.dot(h, wg_ref[l], preferred_element_type=jnp.float32) + bg_ref[l]
        )

        bf = jnp.bfloat16
        rin = (p[:, 2 * U * D:2 * U * D + D] * g[:, 2 * U:2 * U + 1]).astype(bf)
        rout = (p[:, 2 * U * D + D:2 * U * D + 2 * D]
                * g[:, 2 * U + 1:2 * U + 2]).astype(bf)

        rare_bf = adj_rare.astype(bf)
        acc_in = jnp.dot(rare_bf, rin, preferred_element_type=jnp.float32)
        acc_out = jax.lax.dot_general(
            rare_bf, rout, (((0,), (0,)), ((), ())),
            preferred_element_type=jnp.float32)

        for t in range(U):
            hin_t = (p[:, t * D:(t + 1) * D] * g[:, t:t + 1]).astype(bf)
            hout_t = (p[:, (U + t) * D:(U + t + 1) * D]
                      * g[:, U + t:U + t + 1]).astype(bf)
            a_bf = adjb[t].astype(bf)
            acc_in = acc_in + jnp.dot(a_bf, hin_t,
                                      preferred_element_type=jnp.float32)
            acc_out = acc_out + jax.lax.dot_general(
                a_bf, hout_t, (((0,), (0,)), ((), ())),
                preferred_element_type=jnp.float32)

        h = jnp.maximum(acc_in + acc_out, 0.0) + h

    out_ref[0] = h


def kernel(nodes_embed, adj, Win_w, Win_b, Wout_w, Wout_b, Wing_w, Wing_b,
           Woutg_w, Woutg_b, Rin_w, Rin_b, Rout_w, Rout_b, Ring_w, Ring_b,
           Routg_w, Routg_b):
    wio = jnp.concatenate([Win_w, Wout_w, Rin_w, Rout_w], axis=2)
    bio = jnp.concatenate([Win_b, Wout_b, Rin_b, Rout_b], axis=1)
    wg = jnp.concatenate([Wing_w, Woutg_w, Ring_w, Routg_w], axis=2)
    bg = jnp.concatenate([Wing_b, Woutg_b, Ring_b, Routg_b], axis=1)

    return pl.pallas_call(
        _gcnn_kernel,
        grid=(B,),
        in_specs=[
            pl.BlockSpec((1, L, D), lambda b: (b, 0, 0)),
            pl.BlockSpec((1, T, L, L), lambda b: (b, 0, 0, 0)),
            pl.BlockSpec((NB, D, 2 * U * D + 2 * D), lambda b: (0, 0, 0)),
            pl.BlockSpec((NB, 2 * U * D + 2 * D), lambda b: (0, 0)),
            pl.BlockSpec((NB, D, 2 * U + 2), lambda b: (0, 0, 0)),
            pl.BlockSpec((NB, 2 * U + 2), lambda b: (0, 0)),
        ],
        out_specs=pl.BlockSpec((1, L, D), lambda b: (b, 0, 0)),
        out_shape=jax.ShapeDtypeStruct((B, L, D), jnp.float32),
        compiler_params=pltpu.CompilerParams(
            dimension_semantics=("arbitrary",),
            vmem_limit_bytes=100 * 1024 * 1024,
        ),
    )(nodes_embed, adj, wio, bio, wg, bg)


# speedup vs baseline: 2.4856x; 1.1081x over previous
import jax
import jax.numpy as jnp
from jax.experimental import pallas as pl
from jax.experimental.pallas import tpu as pltpu

B, L, D = 4, 512, 140
U, T, NB = 4, 10, 2


def _gcnn_kernel(nodes_ref, adj_ref, wio_ref, bio_ref, wg_ref, bg_ref, out_ref):
    bf = jnp.bfloat16
    h = nodes_ref[0]
    adjb = adj_ref[0]

    adj_rare = adjb[U]
    for t in range(U + 1, T):
        adj_rare = adj_rare + adjb[t]

    a_bf = [adjb[t].astype(bf) for t in range(U)] + [adj_rare.astype(bf)]

    for l in range(NB):
        h_bf = h.astype(bf)
        p = jnp.dot(h_bf, wio_ref[l], preferred_element_type=jnp.float32)
        p = p + bio_ref[l]
        g = jax.nn.sigmoid(
            jnp.dot(h_bf, wg_ref[l], preferred_element_type=jnp.float32)
            + bg_ref[l])

        vin = [(p[:, t * D:(t + 1) * D] * g[:, t:t + 1]).astype(bf)
               for t in range(U)]
        vin.append((p[:, 2 * U * D:2 * U * D + D] * g[:, 2 * U:2 * U + 1])
                   .astype(bf))
        vout = [(p[:, (U + t) * D:(U + t + 1) * D] * g[:, U + t:U + t + 1])
                .astype(bf) for t in range(U)]
        vout.append((p[:, 2 * U * D + D:2 * U * D + 2 * D]
                     * g[:, 2 * U + 1:2 * U + 2]).astype(bf))

        acc_in = jnp.dot(a_bf[0], vin[0], preferred_element_type=jnp.float32)
        for t in range(1, U + 1):
            acc_in = acc_in + jnp.dot(a_bf[t], vin[t],
                                      preferred_element_type=jnp.float32)

        acc_out_t = jax.lax.dot_general(
            vout[0], a_bf[0], (((0,), (0,)), ((), ())),
            preferred_element_type=jnp.float32)
        for t in range(1, U + 1):
            acc_out_t = acc_out_t + jax.lax.dot_general(
                vout[t], a_bf[t], (((0,), (0,)), ((), ())),
                preferred_element_type=jnp.float32)

        h = jnp.maximum(acc_in + acc_out_t.T, 0.0) + h

    out_ref[0] = h


def kernel(nodes_embed, adj, Win_w, Win_b, Wout_w, Wout_b, Wing_w, Wing_b,
           Woutg_w, Woutg_b, Rin_w, Rin_b, Rout_w, Rout_b, Ring_w, Ring_b,
           Routg_w, Routg_b):
    wio = jnp.concatenate([Win_w, Wout_w, Rin_w, Rout_w], axis=2).astype(
        jnp.bfloat16)
    bio = jnp.concatenate([Win_b, Wout_b, Rin_b, Rout_b], axis=1)
    wg = jnp.concatenate([Wing_w, Woutg_w, Ring_w, Routg_w], axis=2).astype(
        jnp.bfloat16)
    bg = jnp.concatenate([Wing_b, Woutg_b, Ring_b, Routg_b], axis=1)

    return pl.pallas_call(
        _gcnn_kernel,
        grid=(B,),
        in_specs=[
            pl.BlockSpec((1, L, D), lambda b: (b, 0, 0)),
            pl.BlockSpec((1, T, L, L), lambda b: (b, 0, 0, 0)),
            pl.BlockSpec((NB, D, 2 * U * D + 2 * D), lambda b: (0, 0, 0)),
            pl.BlockSpec((NB, 2 * U * D + 2 * D), lambda b: (0, 0)),
            pl.BlockSpec((NB, D, 2 * U + 2), lambda b: (0, 0, 0)),
            pl.BlockSpec((NB, 2 * U + 2), lambda b: (0, 0)),
        ],
        out_specs=pl.BlockSpec((1, L, D), lambda b: (b, 0, 0)),
        out_shape=jax.ShapeDtypeStruct((B, L, D), jnp.float32),
        compiler_params=pltpu.CompilerParams(
            dimension_semantics=("arbitrary",),
            vmem_limit_bytes=100 * 1024 * 1024,
        ),
    )(nodes_embed, adj, wio, bio, wg, bg)


# speedup vs baseline: 2.4902x; 1.0018x over previous
import jax
import jax.numpy as jnp
from jax.experimental import pallas as pl
from jax.experimental.pallas import tpu as pltpu

B, L, D = 4, 512, 140
U, T, NB = 4, 10, 2


def _gcnn_kernel(nodes_ref, adj_ref, wio_ref, bio_ref, wg_ref, bg_ref, out_ref):
    bf = jnp.bfloat16
    h = nodes_ref[0]
    adjb = adj_ref[0]

    adj_rare = adjb[U]
    for t in range(U + 1, T):
        adj_rare = adj_rare + adjb[t]

    a_bf = [adjb[t].astype(bf) for t in range(U)] + [adj_rare.astype(bf)]

    for l in range(NB):
        h_bf = h.astype(bf)
        p = jnp.dot(h_bf, wio_ref[l], preferred_element_type=jnp.float32)
        p = p + bio_ref[l]
        g = jax.nn.sigmoid(
            jnp.dot(h_bf, wg_ref[l], preferred_element_type=jnp.float32)
            + bg_ref[l])

        vin = [(p[:, t * D:(t + 1) * D] * g[:, t:t + 1]).astype(bf)
               for t in range(U)]
        vin.append((p[:, 2 * U * D:2 * U * D + D] * g[:, 2 * U:2 * U + 1])
                   .astype(bf))
        vout = [(p[:, (U + t) * D:(U + t + 1) * D] * g[:, U + t:U + t + 1])
                .astype(bf) for t in range(U)]
        vout.append((p[:, 2 * U * D + D:2 * U * D + 2 * D]
                     * g[:, 2 * U + 1:2 * U + 2]).astype(bf))

        acc_in = jnp.dot(a_bf[0], vin[0], preferred_element_type=jnp.float32)
        for t in range(1, U + 1):
            acc_in = acc_in + jnp.dot(a_bf[t], vin[t],
                                      preferred_element_type=jnp.float32)

        acc_out_t = jax.lax.dot_general(
            vout[0], a_bf[0], (((0,), (0,)), ((), ())),
            preferred_element_type=jnp.float32)
        for t in range(1, U + 1):
            acc_out_t = acc_out_t + jax.lax.dot_general(
                vout[t], a_bf[t], (((0,), (0,)), ((), ())),
                preferred_element_type=jnp.float32)

        h = jnp.maximum(acc_in + acc_out_t.T, 0.0) + h

    out_ref[0] = h


def kernel(nodes_embed, adj, Win_w, Win_b, Wout_w, Wout_b, Wing_w, Wing_b,
           Woutg_w, Woutg_b, Rin_w, Rin_b, Rout_w, Rout_b, Ring_w, Ring_b,
           Routg_w, Routg_b):
    wio = jnp.concatenate([Win_w, Wout_w, Rin_w, Rout_w], axis=2).astype(
        jnp.bfloat16)
    bio = jnp.concatenate([Win_b, Wout_b, Rin_b, Rout_b], axis=1)
    wg = jnp.concatenate([Wing_w, Woutg_w, Ring_w, Routg_w], axis=2).astype(
        jnp.bfloat16)
    bg = jnp.concatenate([Wing_b, Woutg_b, Ring_b, Routg_b], axis=1)

    return pl.pallas_call(
        _gcnn_kernel,
        grid=(B,),
        in_specs=[
            pl.BlockSpec((1, L, D), lambda b: (b, 0, 0)),
            pl.BlockSpec((1, T, L, L), lambda b: (b, 0, 0, 0)),
            pl.BlockSpec((NB, D, 2 * U * D + 2 * D), lambda b: (0, 0, 0)),
            pl.BlockSpec((NB, 2 * U * D + 2 * D), lambda b: (0, 0)),
            pl.BlockSpec((NB, D, 2 * U + 2), lambda b: (0, 0, 0)),
            pl.BlockSpec((NB, 2 * U + 2), lambda b: (0, 0)),
        ],
        out_specs=pl.BlockSpec((1, L, D), lambda b: (b, 0, 0)),
        out_shape=jax.ShapeDtypeStruct((B, L, D), jnp.float32),
        compiler_params=pltpu.CompilerParams(
            dimension_semantics=("parallel",),
            vmem_limit_bytes=100 * 1024 * 1024,
        ),
    )(nodes_embed, adj, wio, bio, wg, bg)
